# initial kernel scaffold (unmeasured)
import jax
import jax.numpy as jnp
from jax import lax
from jax.experimental import pallas as pl
from jax.experimental.pallas import tpu as pltpu

N_DEV = 4
K_BLK = 2048


def kernel(x, w_mat):
    m_per, k_dim = x.shape
    _, n_glob = w_mat.shape
    n_per = n_glob // N_DEV
    k_steps = k_dim // K_BLK

    def body(x_ref, w_ref, out_ref, acc, send_buf, send_sems, recv_sems,
             local_sem):
        jj = pl.program_id(0)
        kk = pl.program_id(1)
        my_i = lax.axis_index("i")

        part = jnp.dot(x_ref[...], w_ref[...],
                       preferred_element_type=jnp.float32)

        @pl.when(kk == 0)
        def _():
            acc[...] = part

        @pl.when(kk != 0)
        def _():
            acc[...] = acc[...] + part

        @pl.when(kk == k_steps - 1)
        def _():
            a = acc[...]
            send_buf[jj] = a * jax.nn.sigmoid(a)

            @pl.when(jj == my_i)
            def _():
                pltpu.make_async_copy(
                    send_buf.at[jj],
                    out_ref.at[pl.ds(my_i * m_per, m_per), :],
                    local_sem,
                ).start()

            @pl.when(jj != my_i)
            def _():
                pltpu.make_async_remote_copy(
                    src_ref=send_buf.at[jj],
                    dst_ref=out_ref.at[pl.ds(my_i * m_per, m_per), :],
                    send_sem=send_sems.at[jj],
                    recv_sem=recv_sems.at[my_i],
                    device_id=(jj,),
                    device_id_type=pl.DeviceIdType.MESH,
                ).start()

        @pl.when((jj == N_DEV - 1) & (kk == k_steps - 1))
        def _():
            pltpu.make_async_copy(
                send_buf.at[my_i],
                out_ref.at[pl.ds(my_i * m_per, m_per), :],
                local_sem,
            ).wait()
            for d in range(1, N_DEV):
                t = (my_i + d) % N_DEV
                pltpu.make_async_remote_copy(
                    src_ref=send_buf.at[t],
                    dst_ref=out_ref.at[pl.ds(my_i * m_per, m_per), :],
                    send_sem=send_sems.at[t],
                    recv_sem=recv_sems.at[my_i],
                    device_id=(t,),
                    device_id_type=pl.DeviceIdType.MESH,
                ).wait_send()
            for d in range(1, N_DEV):
                q = (my_i + d) % N_DEV
                pltpu.make_async_remote_copy(
                    src_ref=send_buf.at[q],
                    dst_ref=out_ref.at[pl.ds(q * m_per, m_per), :],
                    send_sem=send_sems.at[q],
                    recv_sem=recv_sems.at[q],
                    device_id=(q,),
                    device_id_type=pl.DeviceIdType.MESH,
                ).wait_recv()

    return pl.pallas_call(
        body,
        grid=(N_DEV, k_steps),
        in_specs=[
            pl.BlockSpec((m_per, K_BLK), lambda j, k: (0, k)),
            pl.BlockSpec((K_BLK, n_per), lambda j, k: (k, j)),
        ],
        out_specs=pl.BlockSpec(memory_space=pltpu.ANY),
        out_shape=jax.ShapeDtypeStruct((N_DEV * m_per, n_per), jnp.float32),
        scratch_shapes=[
            pltpu.VMEM((m_per, n_per), jnp.float32),
            pltpu.VMEM((N_DEV, m_per, n_per), jnp.float32),
            pltpu.SemaphoreType.DMA((N_DEV,)),
            pltpu.SemaphoreType.DMA((N_DEV,)),
            pltpu.SemaphoreType.DMA,
        ],
        compiler_params=pltpu.CompilerParams(
            dimension_semantics=("arbitrary", "arbitrary"),
        ),
    )(x, w_mat)


# baseline (device time: 356410 ns/iter reference)
import jax
import jax.numpy as jnp
from jax import lax
from jax.experimental import pallas as pl
from jax.experimental.pallas import tpu as pltpu

N_DEV = 4
K_BLK = 1024
N_SLOTS = 3


def kernel(x, w_mat):
    m_per, k_dim = x.shape
    _, n_glob = w_mat.shape
    n_per = n_glob // N_DEV
    k_steps = k_dim // K_BLK

    order = (lax.axis_index("i") + jnp.arange(N_DEV, dtype=jnp.int32)) % N_DEV

    def body(ord_ref, x_ref, w_ref, out_ref, send_buf, send_sems, recv_sems,
             local_sem):
        jj = pl.program_id(0)
        kk = pl.program_id(1)
        my_i = lax.axis_index("i")
        tgt = ord_ref[jj]
        slot = lax.rem(jj, N_SLOTS)

        @pl.when((jj == 0) & (kk == 0))
        def _():
            barrier_sem = pltpu.get_barrier_semaphore()
            for d in range(1, N_DEV):
                pl.semaphore_signal(
                    barrier_sem, inc=1,
                    device_id=((my_i + d) % N_DEV,),
                    device_id_type=pl.DeviceIdType.MESH,
                )
            pl.semaphore_wait(barrier_sem, N_DEV - 1)

        @pl.when((jj == N_DEV - 1) & (kk == 0))
        def _():
            pltpu.make_async_copy(
                send_buf.at[0],
                out_ref.at[pl.ds(my_i * m_per, m_per), :],
                local_sem,
            ).wait()

        part = jnp.dot(x_ref[...], w_ref[...],
                       preferred_element_type=jnp.float32)

        @pl.when(kk == 0)
        def _():
            send_buf[slot] = part

        @pl.when(kk != 0)
        def _():
            send_buf[slot] = send_buf[slot] + part

        @pl.when(kk == k_steps - 1)
        def _():
            a = send_buf[slot]
            send_buf[slot] = a * jax.nn.sigmoid(a)

            @pl.when(jj == 0)
            def _():
                pltpu.make_async_copy(
                    send_buf.at[0],
                    out_ref.at[pl.ds(my_i * m_per, m_per), :],
                    local_sem,
                ).start()

            @pl.when(jj != 0)
            def _():
                pltpu.make_async_remote_copy(
                    src_ref=send_buf.at[slot],
                    dst_ref=out_ref.at[pl.ds(my_i * m_per, m_per), :],
                    send_sem=send_sems.at[jj],
                    recv_sem=recv_sems.at[my_i],
                    device_id=(tgt,),
                    device_id_type=pl.DeviceIdType.MESH,
                ).start()

        @pl.when((jj == N_DEV - 1) & (kk == k_steps - 1))
        def _():
            for d in range(1, N_DEV):
                pltpu.make_async_remote_copy(
                    src_ref=send_buf.at[d % N_SLOTS],
                    dst_ref=out_ref.at[pl.ds(my_i * m_per, m_per), :],
                    send_sem=send_sems.at[d],
                    recv_sem=recv_sems.at[my_i],
                    device_id=((my_i + d) % N_DEV,),
                    device_id_type=pl.DeviceIdType.MESH,
                ).wait_send()
            for d in range(1, N_DEV):
                q = (my_i + d) % N_DEV
                pltpu.make_async_remote_copy(
                    src_ref=send_buf.at[d % N_SLOTS],
                    dst_ref=out_ref.at[pl.ds(q * m_per, m_per), :],
                    send_sem=send_sems.at[d],
                    recv_sem=recv_sems.at[q],
                    device_id=(q,),
                    device_id_type=pl.DeviceIdType.MESH,
                ).wait_recv()

    grid_spec = pltpu.PrefetchScalarGridSpec(
        num_scalar_prefetch=1,
        grid=(N_DEV, k_steps),
        in_specs=[
            pl.BlockSpec((m_per, K_BLK), lambda j, k, ord_ref: (0, k)),
            pl.BlockSpec((K_BLK, n_per), lambda j, k, ord_ref: (k, ord_ref[j])),
        ],
        out_specs=pl.BlockSpec(memory_space=pl.ANY),
        scratch_shapes=[
            pltpu.VMEM((N_SLOTS, m_per, n_per), jnp.float32),
            pltpu.SemaphoreType.DMA((N_DEV,)),
            pltpu.SemaphoreType.DMA((N_DEV,)),
            pltpu.SemaphoreType.DMA,
        ],
    )

    return pl.pallas_call(
        body,
        grid_spec=grid_spec,
        out_shape=jax.ShapeDtypeStruct((N_DEV * m_per, n_per), jnp.float32),
        compiler_params=pltpu.CompilerParams(
            dimension_semantics=("arbitrary", "arbitrary"),
            vmem_limit_bytes=60 * 1024 * 1024,
            collective_id=0,
        ),
    )(order, x, w_mat)
